# baseline (speedup 1.0000x reference)
import functools

import numpy as np
import jax
import jax.numpy as jnp
from jax import lax
from jax.experimental import pallas as pl
from jax.experimental.pallas import tpu as pltpu
from jax.experimental.pallas import tpu_sc as plsc

_N = 256
_K = 128
_BATCH = 16384
_BLOCK = 1024


def _butterfly_np(u):
    n_cur = u.shape[1]
    big_v = [u]
    num_of_splits = 1
    v = u
    while n_cur > 1:
        v_odd = np.concatenate([w[:, 0::2] for w in big_v], axis=1)
        v_even = np.concatenate([w[:, 1::2] for w in big_v], axis=1)
        v_xor = (v_odd + v_even) % 2
        xs = np.split(v_xor, 2 ** (num_of_splits - 1), axis=1)
        ids = np.split(v_even, 2 ** (num_of_splits - 1), axis=1)
        v = np.concatenate([e for pair in zip(xs, ids) for e in pair], axis=1)
        big_v = np.split(v, 2 ** num_of_splits, axis=1)
        n_cur //= 2
        num_of_splits += 1
    return v


_G_NP = _butterfly_np(np.eye(_N, dtype=np.int64)).astype(np.float32)


def _threefry2x32_np(k0, k1, x0, x1):
    rot = ((13, 15, 26, 6), (17, 29, 16, 24))
    ks = (np.uint32(k0), np.uint32(k1),
          np.uint32(0x1BD11BDA) ^ np.uint32(k0) ^ np.uint32(k1))
    x0 = (x0 + ks[0]).astype(np.uint32)
    x1 = (x1 + ks[1]).astype(np.uint32)
    for i in range(5):
        for r in rot[i % 2]:
            x0 = (x0 + x1).astype(np.uint32)
            x1 = ((x1 << np.uint32(r)) | (x1 >> np.uint32(32 - r))).astype(np.uint32)
            x1 = x1 ^ x0
        x0 = (x0 + ks[(i + 1) % 3]).astype(np.uint32)
        x1 = (x1 + ks[(i + 2) % 3] + np.uint32(i + 1)).astype(np.uint32)
    return x0, x1


@functools.lru_cache(maxsize=2)
def _rand8_np(batch):
    err = np.seterr(over="ignore")
    try:
        size = batch * _N
        kd = (np.uint32(0), np.uint32(42))
        if jax.config.jax_threefry_partitionable:
            s0, s1 = _threefry2x32_np(kd[0], kd[1], np.zeros(2, np.uint32),
                                      np.arange(2, dtype=np.uint32))
            k2 = (s0[1], s1[1])
            idx = np.arange(size, dtype=np.uint64)
            hi = (idx >> np.uint64(32)).astype(np.uint32)
            lo = (idx & np.uint64(0xFFFFFFFF)).astype(np.uint32)
            b0, b1 = _threefry2x32_np(k2[0], k2[1], hi, lo)
            bits = b0 ^ b1
        else:
            c = np.arange(4, dtype=np.uint32)
            y0, y1 = _threefry2x32_np(kd[0], kd[1], c[:2], c[2:])
            k2 = np.concatenate([y0, y1]).reshape(2, 2)[1]
            c = np.arange(size, dtype=np.uint32)
            b0, b1 = _threefry2x32_np(k2[0], k2[1], c[: size // 2], c[size // 2:])
            bits = np.concatenate([b0, b1])
        return (bits & np.uint32(1)).astype(np.int8).reshape(batch, _N)
    finally:
        np.seterr(**err)


def _half_sc(batch):
    info = plsc.get_sparse_core_info()
    nw = info.num_cores * info.num_subcores
    rows = 4 * batch
    chunk = 128
    per_w = rows // nw
    n_chunks = per_w // chunk
    mesh = plsc.VectorSubcoreMesh(core_axis_name="c", subcore_axis_name="s")

    @functools.partial(
        pl.kernel,
        mesh=mesh,
        out_type=jax.ShapeDtypeStruct((rows, _K), jnp.float32),
        scratch_types=[pltpu.VMEM((chunk, _K), jnp.float32)],
    )
    def k(out_hbm, buf):
        half16 = jnp.full((16,), 0.5, dtype=jnp.float32)

        def fill_row(i, _):
            for j in range(_K // 16):
                buf[i, pl.ds(16 * j, 16)] = half16
            return 0

        lax.fori_loop(0, chunk, fill_row, 0)
        wid = lax.axis_index("s") * info.num_cores + lax.axis_index("c")
        base = wid * per_w

        def copy_chunk(c, _):
            pltpu.sync_copy(buf, out_hbm.at[pl.ds(base + c * chunk, chunk), :])
            return 0

        lax.fori_loop(0, n_chunks, copy_chunk, 0)

    return k()


def _body(info_ref, rand8_ref, g_ref, x_ref, u_ref, f_ref, r_ref):
    info = info_ref[...]
    rand8 = rand8_ref[...]
    b = info.shape[0]
    rand_right = rand8[:, _K:].astype(jnp.int32)
    u_ref[pl.Slice(0, b, 2), :] = info
    u_ref[pl.Slice(1, b, 2), :] = rand_right
    f_ref[pl.Slice(0, b, 2), :] = jnp.full_like(info, 2)
    f_ref[pl.Slice(1, b, 2), :] = rand_right

    u2 = jnp.concatenate([info, rand_right], axis=1)
    acc = jnp.dot(u2.astype(jnp.bfloat16), g_ref[...],
                  preferred_element_type=jnp.float32)
    xb = jnp.bitwise_and(acc.astype(jnp.int32), 1)
    x_ref[pl.Slice(0, b, 2), :] = xb[:, :_K]
    x_ref[pl.Slice(1, b, 2), :] = xb[:, _K:]

    uf = rand8.astype(jnp.float32)
    r_ref[pl.Slice(0, b, 4), :] = 1.0 - uf[:, :_K]
    r_ref[pl.Slice(1, b, 4), :] = uf[:, :_K]
    r_ref[pl.Slice(2, b, 4), :] = 1.0 - uf[:, _K:]
    r_ref[pl.Slice(3, b, 4), :] = uf[:, _K:]


def _run(info_bits, rand8, block):
    batch = info_bits.shape[0]
    grid = batch // block
    g = jnp.asarray(_G_NP, dtype=jnp.bfloat16)
    out_shapes = (
        jax.ShapeDtypeStruct((2 * batch, _K), jnp.int32),
        jax.ShapeDtypeStruct((2 * batch, _K), jnp.int32),
        jax.ShapeDtypeStruct((2 * batch, _K), jnp.int32),
        jax.ShapeDtypeStruct((4 * batch, _K), jnp.float32),
    )
    spec = lambda rows, cols: pl.BlockSpec((rows, cols), lambda i: (i, 0))
    return pl.pallas_call(
        _body,
        grid=(grid,),
        in_specs=[
            spec(block, _K),
            spec(block, _N),
            pl.BlockSpec((_N, _N), lambda i: (0, 0)),
        ],
        out_specs=(
            spec(2 * block, _K),
            spec(2 * block, _K),
            spec(2 * block, _K),
            spec(4 * block, _K),
        ),
        out_shape=out_shapes,
    )(info_bits, rand8, g)


def kernel(inputs):
    info_bits = inputs
    batch = info_bits.shape[0]
    rand8 = jnp.asarray(_rand8_np(batch))
    block = _BLOCK if batch % _BLOCK == 0 else batch
    x2, u2, f2, r2 = _run(info_bits, rand8, block)
    if (4 * batch) % 4096 == 0:
        half2 = _half_sc(batch)
    else:
        half2 = jnp.full((4 * batch, _K), 0.5, dtype=jnp.float32)
    x = x2.reshape(batch, _N, 1)
    u = u2.reshape(batch, _N, 1)
    f = f2.reshape(batch, _N, 1)

    def _pairs(a):
        return a.reshape(batch, 2, 2, _K).transpose(0, 1, 3, 2).reshape(batch, _N, 2)

    half = _pairs(half2)
    r = _pairs(r2)
    return (x, f, u, half, r)


# speedup vs baseline: 1.0237x; 1.0237x over previous
import functools

import numpy as np
import jax
import jax.numpy as jnp
from jax import lax
from jax.experimental import pallas as pl
from jax.experimental.pallas import tpu as pltpu
from jax.experimental.pallas import tpu_sc as plsc

_N = 256
_K = 128
_BATCH = 16384
_BLOCK = 2048


def _butterfly_np(u):
    n_cur = u.shape[1]
    big_v = [u]
    num_of_splits = 1
    v = u
    while n_cur > 1:
        v_odd = np.concatenate([w[:, 0::2] for w in big_v], axis=1)
        v_even = np.concatenate([w[:, 1::2] for w in big_v], axis=1)
        v_xor = (v_odd + v_even) % 2
        xs = np.split(v_xor, 2 ** (num_of_splits - 1), axis=1)
        ids = np.split(v_even, 2 ** (num_of_splits - 1), axis=1)
        v = np.concatenate([e for pair in zip(xs, ids) for e in pair], axis=1)
        big_v = np.split(v, 2 ** num_of_splits, axis=1)
        n_cur //= 2
        num_of_splits += 1
    return v


_G_NP = _butterfly_np(np.eye(_N, dtype=np.int64)).astype(np.float32)


def _threefry2x32_np(k0, k1, x0, x1):
    rot = ((13, 15, 26, 6), (17, 29, 16, 24))
    ks = (np.uint32(k0), np.uint32(k1),
          np.uint32(0x1BD11BDA) ^ np.uint32(k0) ^ np.uint32(k1))
    x0 = (x0 + ks[0]).astype(np.uint32)
    x1 = (x1 + ks[1]).astype(np.uint32)
    for i in range(5):
        for r in rot[i % 2]:
            x0 = (x0 + x1).astype(np.uint32)
            x1 = ((x1 << np.uint32(r)) | (x1 >> np.uint32(32 - r))).astype(np.uint32)
            x1 = x1 ^ x0
        x0 = (x0 + ks[(i + 1) % 3]).astype(np.uint32)
        x1 = (x1 + ks[(i + 2) % 3] + np.uint32(i + 1)).astype(np.uint32)
    return x0, x1


@functools.lru_cache(maxsize=2)
def _rand8_np(batch):
    err = np.seterr(over="ignore")
    try:
        size = batch * _N
        kd = (np.uint32(0), np.uint32(42))
        if jax.config.jax_threefry_partitionable:
            s0, s1 = _threefry2x32_np(kd[0], kd[1], np.zeros(2, np.uint32),
                                      np.arange(2, dtype=np.uint32))
            k2 = (s0[1], s1[1])
            idx = np.arange(size, dtype=np.uint64)
            hi = (idx >> np.uint64(32)).astype(np.uint32)
            lo = (idx & np.uint64(0xFFFFFFFF)).astype(np.uint32)
            b0, b1 = _threefry2x32_np(k2[0], k2[1], hi, lo)
            bits = b0 ^ b1
        else:
            c = np.arange(4, dtype=np.uint32)
            y0, y1 = _threefry2x32_np(kd[0], kd[1], c[:2], c[2:])
            k2 = np.concatenate([y0, y1]).reshape(2, 2)[1]
            c = np.arange(size, dtype=np.uint32)
            b0, b1 = _threefry2x32_np(k2[0], k2[1], c[: size // 2], c[size // 2:])
            bits = np.concatenate([b0, b1])
        return (bits & np.uint32(1)).astype(np.int8).reshape(batch, _N)
    finally:
        np.seterr(**err)


def _half_sc(batch):
    info = plsc.get_sparse_core_info()
    nw = info.num_cores * info.num_subcores
    rows = 4 * batch
    chunk = 128
    per_w = rows // nw
    n_chunks = per_w // chunk
    mesh = plsc.VectorSubcoreMesh(core_axis_name="c", subcore_axis_name="s")

    @functools.partial(
        pl.kernel,
        mesh=mesh,
        out_type=jax.ShapeDtypeStruct((rows, _K), jnp.float32),
        scratch_types=[pltpu.VMEM((chunk, _K), jnp.float32),
                       pltpu.SemaphoreType.DMA],
    )
    def k(out_hbm, buf, sem):
        half16 = jnp.full((16,), 0.5, dtype=jnp.float32)

        def fill_row(i, _):
            for j in range(_K // 16):
                buf[i, pl.ds(16 * j, 16)] = half16
            return 0

        lax.fori_loop(0, chunk, fill_row, 0)
        wid = lax.axis_index("s") * info.num_cores + lax.axis_index("c")
        base = wid * per_w
        handles = [
            pltpu.async_copy(buf, out_hbm.at[pl.ds(base + c * chunk, chunk), :], sem)
            for c in range(n_chunks)
        ]
        for h in handles:
            h.wait()

    return k()


def _body(info_ref, rand8_ref, g_ref, x_ref, u_ref, f_ref, r_ref):
    info = info_ref[...]
    rand8 = rand8_ref[...]
    b = info.shape[0]
    rand_right = rand8[:, _K:].astype(jnp.int32)
    u_ref[pl.Slice(0, b, 2), :] = info
    u_ref[pl.Slice(1, b, 2), :] = rand_right
    f_ref[pl.Slice(0, b, 2), :] = jnp.full_like(info, 2)
    f_ref[pl.Slice(1, b, 2), :] = rand_right

    u2 = jnp.concatenate([info, rand_right], axis=1)
    acc = jnp.dot(u2.astype(jnp.bfloat16), g_ref[...],
                  preferred_element_type=jnp.float32)
    xb = jnp.bitwise_and(acc.astype(jnp.int32), 1)
    x_ref[pl.Slice(0, b, 2), :] = xb[:, :_K]
    x_ref[pl.Slice(1, b, 2), :] = xb[:, _K:]

    uf = rand8.astype(jnp.float32)
    r_ref[pl.Slice(0, b, 4), :] = 1.0 - uf[:, :_K]
    r_ref[pl.Slice(1, b, 4), :] = uf[:, :_K]
    r_ref[pl.Slice(2, b, 4), :] = 1.0 - uf[:, _K:]
    r_ref[pl.Slice(3, b, 4), :] = uf[:, _K:]


def _run(info_bits, rand8, block):
    batch = info_bits.shape[0]
    grid = batch // block
    g = jnp.asarray(_G_NP, dtype=jnp.bfloat16)
    out_shapes = (
        jax.ShapeDtypeStruct((2 * batch, _K), jnp.int32),
        jax.ShapeDtypeStruct((2 * batch, _K), jnp.int32),
        jax.ShapeDtypeStruct((2 * batch, _K), jnp.int32),
        jax.ShapeDtypeStruct((4 * batch, _K), jnp.float32),
    )
    spec = lambda rows, cols: pl.BlockSpec((rows, cols), lambda i: (i, 0))
    return pl.pallas_call(
        _body,
        grid=(grid,),
        in_specs=[
            spec(block, _K),
            spec(block, _N),
            pl.BlockSpec((_N, _N), lambda i: (0, 0)),
        ],
        out_specs=(
            spec(2 * block, _K),
            spec(2 * block, _K),
            spec(2 * block, _K),
            spec(4 * block, _K),
        ),
        out_shape=out_shapes,
    )(info_bits, rand8, g)


def kernel(inputs):
    info_bits = inputs
    batch = info_bits.shape[0]
    rand8 = jnp.asarray(_rand8_np(batch))
    block = _BLOCK if batch % _BLOCK == 0 else batch
    x2, u2, f2, r2 = _run(info_bits, rand8, block)
    if (4 * batch) % 4096 == 0:
        half2 = _half_sc(batch)
    else:
        half2 = jnp.full((4 * batch, _K), 0.5, dtype=jnp.float32)
    x = x2.reshape(batch, _N, 1)
    u = u2.reshape(batch, _N, 1)
    f = f2.reshape(batch, _N, 1)

    def _pairs(a):
        return a.reshape(batch, 2, 2, _K).transpose(0, 1, 3, 2).reshape(batch, _N, 2)

    half = _pairs(half2)
    r = _pairs(r2)
    return (x, f, u, half, r)


# speedup vs baseline: 1.4100x; 1.3773x over previous
import functools

import numpy as np
import jax
import jax.numpy as jnp
from jax.experimental import pallas as pl

_N = 256
_K = 128
_BATCH = 16384
_BLOCK = 2048


def _butterfly_np(u):
    n_cur = u.shape[1]
    big_v = [u]
    num_of_splits = 1
    v = u
    while n_cur > 1:
        v_odd = np.concatenate([w[:, 0::2] for w in big_v], axis=1)
        v_even = np.concatenate([w[:, 1::2] for w in big_v], axis=1)
        v_xor = (v_odd + v_even) % 2
        xs = np.split(v_xor, 2 ** (num_of_splits - 1), axis=1)
        ids = np.split(v_even, 2 ** (num_of_splits - 1), axis=1)
        v = np.concatenate([e for pair in zip(xs, ids) for e in pair], axis=1)
        big_v = np.split(v, 2 ** num_of_splits, axis=1)
        n_cur //= 2
        num_of_splits += 1
    return v


_G_NP = _butterfly_np(np.eye(_N, dtype=np.int64)).astype(np.float32)


def _threefry2x32_np(k0, k1, x0, x1):
    rot = ((13, 15, 26, 6), (17, 29, 16, 24))
    ks = (np.uint32(k0), np.uint32(k1),
          np.uint32(0x1BD11BDA) ^ np.uint32(k0) ^ np.uint32(k1))
    x0 = (x0 + ks[0]).astype(np.uint32)
    x1 = (x1 + ks[1]).astype(np.uint32)
    for i in range(5):
        for r in rot[i % 2]:
            x0 = (x0 + x1).astype(np.uint32)
            x1 = ((x1 << np.uint32(r)) | (x1 >> np.uint32(32 - r))).astype(np.uint32)
            x1 = x1 ^ x0
        x0 = (x0 + ks[(i + 1) % 3]).astype(np.uint32)
        x1 = (x1 + ks[(i + 2) % 3] + np.uint32(i + 1)).astype(np.uint32)
    return x0, x1


@functools.lru_cache(maxsize=2)
def _rand8_np(batch):
    err = np.seterr(over="ignore")
    try:
        size = batch * _N
        kd = (np.uint32(0), np.uint32(42))
        if jax.config.jax_threefry_partitionable:
            s0, s1 = _threefry2x32_np(kd[0], kd[1], np.zeros(2, np.uint32),
                                      np.arange(2, dtype=np.uint32))
            k2 = (s0[1], s1[1])
            idx = np.arange(size, dtype=np.uint64)
            hi = (idx >> np.uint64(32)).astype(np.uint32)
            lo = (idx & np.uint64(0xFFFFFFFF)).astype(np.uint32)
            b0, b1 = _threefry2x32_np(k2[0], k2[1], hi, lo)
            bits = b0 ^ b1
        else:
            c = np.arange(4, dtype=np.uint32)
            y0, y1 = _threefry2x32_np(kd[0], kd[1], c[:2], c[2:])
            k2 = np.concatenate([y0, y1]).reshape(2, 2)[1]
            c = np.arange(size, dtype=np.uint32)
            b0, b1 = _threefry2x32_np(k2[0], k2[1], c[: size // 2], c[size // 2:])
            bits = np.concatenate([b0, b1])
        return (bits & np.uint32(1)).astype(np.int8).reshape(batch, _N)
    finally:
        np.seterr(**err)


def _body(info_ref, rand8_ref, g_ref, x_ref, u_ref, f_ref, half_ref, r_ref):
    info = info_ref[...]
    rand8 = rand8_ref[...]
    b = info.shape[0]
    rand_right = rand8[:, _K:].astype(jnp.int32)
    u_ref[pl.Slice(0, b, 2), :] = info
    u_ref[pl.Slice(1, b, 2), :] = rand_right
    f_ref[pl.Slice(0, b, 2), :] = jnp.full_like(info, 2)
    f_ref[pl.Slice(1, b, 2), :] = rand_right

    u2 = jnp.concatenate([info, rand_right], axis=1)
    acc = jnp.dot(u2.astype(jnp.bfloat16), g_ref[...],
                  preferred_element_type=jnp.float32)
    xb = jnp.bitwise_and(acc.astype(jnp.int32), 1)
    x_ref[pl.Slice(0, b, 2), :] = xb[:, :_K]
    x_ref[pl.Slice(1, b, 2), :] = xb[:, _K:]

    half_ref[...] = jnp.full_like(half_ref, 0.5)

    uf = rand8.astype(jnp.float32)
    r_ref[pl.Slice(0, b, 4), :] = 1.0 - uf[:, :_K]
    r_ref[pl.Slice(1, b, 4), :] = uf[:, :_K]
    r_ref[pl.Slice(2, b, 4), :] = 1.0 - uf[:, _K:]
    r_ref[pl.Slice(3, b, 4), :] = uf[:, _K:]


def _run(info_bits, rand8, block):
    batch = info_bits.shape[0]
    grid = batch // block
    g = jnp.asarray(_G_NP, dtype=jnp.bfloat16)
    out_shapes = (
        jax.ShapeDtypeStruct((2 * batch, _K), jnp.int32),
        jax.ShapeDtypeStruct((2 * batch, _K), jnp.int32),
        jax.ShapeDtypeStruct((2 * batch, _K), jnp.int32),
        jax.ShapeDtypeStruct((4 * batch, _K), jnp.float32),
        jax.ShapeDtypeStruct((4 * batch, _K), jnp.float32),
    )
    spec = lambda rows, cols: pl.BlockSpec((rows, cols), lambda i: (i, 0))
    return pl.pallas_call(
        _body,
        grid=(grid,),
        in_specs=[
            spec(block, _K),
            spec(block, _N),
            pl.BlockSpec((_N, _N), lambda i: (0, 0)),
        ],
        out_specs=(
            spec(2 * block, _K),
            spec(2 * block, _K),
            spec(2 * block, _K),
            spec(4 * block, _K),
            spec(4 * block, _K),
        ),
        out_shape=out_shapes,
    )(info_bits, rand8, g)


def kernel(inputs):
    info_bits = inputs
    batch = info_bits.shape[0]
    rand8 = jnp.asarray(_rand8_np(batch))
    block = _BLOCK if batch % _BLOCK == 0 else batch
    x2, u2, f2, half2, r2 = _run(info_bits, rand8, block)
    x = x2.reshape(batch, _N, 1)
    u = u2.reshape(batch, _N, 1)
    f = f2.reshape(batch, _N, 1)

    def _pairs(a):
        return a.reshape(batch, 2, 2, _K).transpose(0, 1, 3, 2).reshape(batch, _N, 2)

    half = _pairs(half2)
    r = _pairs(r2)
    return (x, f, u, half, r)
